# Initial kernel scaffold; baseline (speedup 1.0000x reference)
import jax
import jax.numpy as jnp
from jax.experimental import pallas as pl


def kernel(x, edge_index, batch, W1s, b1s, g1s, bt1s, W2s, b2s, lin_W, lin_b):
    raise NotImplementedError("write your pallas kernel here")


# speedup vs baseline: 7.4053x; 7.4053x over previous
import functools

import jax
import jax.numpy as jnp
from jax import lax
from jax.experimental import pallas as pl
from jax.experimental.pallas import tpu as pltpu
from jax.experimental.pallas import tpu_sc as plsc

N_NODES = 10000
N_EDGES = 320000
D = 128
NUM_GRAPHS = 64

NC = 2
NS = 16
NW = NC * NS
EPW = N_EDGES // NW
CH = 125
NCH = EPW // CH
N_PAD = 10240
ROWS_PER_TILE = N_PAD // NS

@functools.cache
def _make_sc_scatter():
    mesh = plsc.VectorSubcoreMesh(core_axis_name="c", subcore_axis_name="s")

    @functools.partial(
        pl.kernel,
        mesh=mesh,
        out_type=jax.ShapeDtypeStruct((NC, N_PAD, D), jnp.float32),
        scratch_types=[
            pltpu.VMEM((NCH, CH), jnp.int32),
            pltpu.VMEM((NCH, CH), jnp.int32),
            pltpu.VMEM((CH, D), jnp.float32),
            pltpu.VMEM_SHARED((N_PAD, D), jnp.float32),
            pltpu.SemaphoreType.DMA,
        ],
    )
    def _sc_scatter(h_hbm, src_hbm, dst_hbm, zeros_hbm, out_hbm,
                    src_v, dst_v, rows_v, acc, sem):
        c = lax.axis_index("c")
        s = lax.axis_index("s")
        wid = s * NC + c

        if True:
            pltpu.sync_copy(zeros_hbm,
                            acc.at[pl.ds(s * ROWS_PER_TILE, ROWS_PER_TILE)])
            pltpu.sync_copy(src_hbm.at[wid], src_v)
            pltpu.sync_copy(dst_hbm.at[wid], dst_v)
            plsc.subcore_barrier()

            def chunk(j, carry):
                pltpu.async_copy(h_hbm.at[src_v.at[j]], rows_v, sem).wait()
                pltpu.sync_copy(rows_v, acc.at[dst_v.at[j]], add=True)
                return carry

            lax.fori_loop(0, NCH, chunk, 0)
            plsc.subcore_barrier()
            pltpu.sync_copy(acc.at[pl.ds(s * ROWS_PER_TILE, ROWS_PER_TILE)],
                            out_hbm.at[c, pl.ds(s * ROWS_PER_TILE, ROWS_PER_TILE)])

    return _sc_scatter


def _dense_common(h_ref, agg_ref, W1_ref, b1_ref, g_ref, bt_ref, W2_ref, b2_ref):
    hin = h_ref[:] + agg_ref[0, :N_NODES] + agg_ref[1, :N_NODES]
    t = jnp.dot(hin, W1_ref[:], preferred_element_type=jnp.float32) + b1_ref[:]
    t = jnp.maximum(t, 0.0)
    mu = jnp.mean(t, axis=0, keepdims=True)
    var = jnp.mean((t - mu) ** 2, axis=0, keepdims=True)
    t = (t - mu) / jnp.sqrt(var + 1e-5) * g_ref[:] + bt_ref[:]
    t = jnp.dot(t, W2_ref[:], preferred_element_type=jnp.float32) + b2_ref[:]
    return jnp.maximum(t, 0.0)


def _dense_body(h_ref, agg_ref, W1_ref, b1_ref, g_ref, bt_ref, W2_ref, b2_ref, o_ref):
    o_ref[:] = _dense_common(h_ref, agg_ref, W1_ref, b1_ref, g_ref, bt_ref,
                             W2_ref, b2_ref)


def _dense_res_body(h_ref, agg_ref, W1_ref, b1_ref, g_ref, bt_ref, W2_ref, b2_ref,
                    res_ref, o_ref):
    o_ref[:] = _dense_common(h_ref, agg_ref, W1_ref, b1_ref, g_ref, bt_ref,
                             W2_ref, b2_ref) + res_ref[:]


_dense = pl.pallas_call(
    _dense_body, out_shape=jax.ShapeDtypeStruct((N_NODES, D), jnp.float32))
_dense_res = pl.pallas_call(
    _dense_res_body, out_shape=jax.ShapeDtypeStruct((N_NODES, D), jnp.float32))


def _pool_body(h_ref, batch_ref, linW_ref, linb_ref, o_ref):
    onehot = (lax.broadcasted_iota(jnp.int32, (NUM_GRAPHS, N_NODES), 0)
              == batch_ref[:]).astype(jnp.float32)
    sums = jnp.dot(onehot, h_ref[:], preferred_element_type=jnp.float32)
    counts = jnp.sum(onehot, axis=1, keepdims=True)
    pooled = sums / jnp.maximum(counts, 1.0)
    o_ref[:] = jnp.dot(pooled, linW_ref[:],
                       preferred_element_type=jnp.float32) + linb_ref[:]


_pool = pl.pallas_call(
    _pool_body, out_shape=jax.ShapeDtypeStruct((NUM_GRAPHS, 1), jnp.float32))


def kernel(x, edge_index, batch, W1s, b1s, g1s, bt1s, W2s, b2s, lin_W, lin_b):
    src = edge_index[0].astype(jnp.int32).reshape(NW, NCH, CH)
    dst = edge_index[1].astype(jnp.int32).reshape(NW, NCH, CH)
    zeros = jnp.zeros((ROWS_PER_TILE, D), jnp.float32)
    batch2 = batch.astype(jnp.int32).reshape(1, N_NODES)

    sc_scatter = _make_sc_scatter()
    h = x
    for blk in range(3):
        h0 = h
        for li in (2 * blk, 2 * blk + 1):
            agg = sc_scatter(h, src, dst, zeros)
            args = (h, agg, W1s[li], b1s[li].reshape(1, D), g1s[li].reshape(1, D),
                    bt1s[li].reshape(1, D), W2s[li], b2s[li].reshape(1, D))
            if li % 2 == 1:
                h = _dense_res(*args, h0)
            else:
                h = _dense(*args)
    return _pool(h, batch2, lin_W, lin_b.reshape(1, 1))
